# baseline (speedup 1.0000x reference)
import functools

import jax
import jax.numpy as jnp
from jax import lax
from jax.experimental import pallas as pl
from jax.experimental.pallas import tpu as pltpu
from jax.experimental.pallas import tpu_sc as plsc

N = 10000
DIM = 512
NUM_SEG = 512
LAYERS = 6
LANES = 16

NC = 2
NS = 16
NW = NC * NS

CHUNK = 160
SG = 8
CG = 4
CW = DIM // CG
SEGT = NUM_SEG // SG
BND_PAD = SG * SEGT + SEGT + LANES

CH3 = 40
NCH3 = N // CH3
NFULL3 = NCH3 // NW
NTAIL3 = NCH3 - NW * NFULL3

_mesh = plsc.VectorSubcoreMesh(core_axis_name="c", subcore_axis_name="s")


@functools.partial(
    pl.kernel,
    out_type=jax.ShapeDtypeStruct((NUM_SEG, DIM), jnp.float32),
    mesh=_mesh,
    scratch_types=[
        pltpu.VMEM((CHUNK, CW), jnp.float32),
        pltpu.VMEM((CHUNK, CW), jnp.float32),
        pltpu.VMEM((SEGT + LANES,), jnp.int32),
        pltpu.VMEM((CHUNK + LANES,), jnp.int32),
        pltpu.VMEM((CHUNK + LANES,), jnp.int32),
        pltpu.VMEM((SEGT, CW), jnp.float32),
        pltpu.SemaphoreType.DMA,
        pltpu.SemaphoreType.DMA,
        pltpu.SemaphoreType.DMA,
        pltpu.SemaphoreType.DMA,
        pltpu.SemaphoreType.DMA,
    ],
)
def _sc_scatter(x_hbm, bnd_hbm, zeros_hbm, idxf_hbm, out_hbm,
                rows_v0, rows_v1, bnd_v, idx_v0, idx_v1, acc_v,
                s_r0, s_r1, s_i0, s_i1, s_z):
    cid = lax.axis_index("c")
    sid = lax.axis_index("s")
    wid = sid * NC + cid
    sg = wid // CG
    cg = wid % CG
    c0 = cg * CW

    zc = pltpu.async_copy(zeros_hbm.at[pl.ds(0, SEGT), pl.ds(c0, CW)],
                          acc_v, s_z)
    pltpu.sync_copy(bnd_hbm.at[pl.ds(sg * SEGT, SEGT + LANES)], bnd_v)
    zc.wait()

    def bget(i):
        return bnd_v[pl.ds(i, LANES)][0]

    rlo = bget(0)
    rhi = bget(SEGT)
    start = (rlo // 8) * 8

    zacc = tuple(jnp.zeros((LANES,), jnp.float32) for _ in range(CW // LANES))

    rows_b = (rows_v0, rows_v1)
    idx_b = (idx_v0, idx_v1)
    s_r = (s_r0, s_r1)
    s_i = (s_i0, s_i1)

    def base_of(cstart):
        return (jnp.minimum(cstart, N - CHUNK) // 8) * 8

    def start_chunk(cstart, bi):
        bk = base_of(cstart)
        pltpu.async_copy(x_hbm.at[pl.ds(bk, CHUNK), pl.ds(c0, CW)],
                         rows_b[bi], s_r[bi])
        pltpu.async_copy(idxf_hbm.at[pl.ds(bk, CHUNK)],
                         idx_b[bi].at[pl.ds(0, CHUNK)], s_i[bi])

    def wait_chunk(cstart, bi):
        bk = base_of(cstart)
        pltpu.make_async_copy(x_hbm.at[pl.ds(bk, CHUNK), pl.ds(c0, CW)],
                              rows_b[bi], s_r[bi]).wait()
        pltpu.make_async_copy(idxf_hbm.at[pl.ds(bk, CHUNK)],
                              idx_b[bi].at[pl.ds(0, CHUNK)], s_i[bi]).wait()

    def add_range(accs, lo, hi, bk, bi):
        def body(r, a):
            return tuple(
                a[j] + rows_b[bi][r - bk, pl.ds(j * LANES, LANES)]
                for j in range(CW // LANES)
            )
        return lax.fori_loop(lo, hi, body, accs)

    def flush_add(s_local, accs):
        for j in range(CW // LANES):
            sl = pl.ds(j * LANES, LANES)
            acc_v[s_local, sl] = acc_v[s_local, sl] + accs[j]

    def proc_chunk(cstart, s, bi):
        bk = base_of(cstart)
        win_hi = jnp.minimum(cstart + CHUNK, rhi)

        s_hi = idx_b[bi][pl.ds(win_hi - 1 - bk, LANES)][0] - sg * SEGT

        def seg_body(s2, carry):
            accs = add_range(zacc, jnp.maximum(bget(s2), cstart),
                             bget(s2 + 1), bk, bi)
            flush_add(s2, accs)
            return carry

        lax.fori_loop(s, s_hi, seg_body, 0)
        accs3 = add_range(zacc, jnp.maximum(bget(s_hi), cstart), win_hi,
                          bk, bi)
        flush_add(jnp.clip(s_hi, 0, SEGT - 1), accs3)
        return s_hi

    @pl.when(start < rhi)
    def _prologue():
        start_chunk(start, 0)

    def pair_step(p, s):
        cs0 = start + (2 * p) * CHUNK
        cs1 = cs0 + CHUNK
        cs2 = cs1 + CHUNK

        def do_pair(s):
            @pl.when(cs1 < rhi)
            def _pf1():
                start_chunk(cs1, 1)

            wait_chunk(cs0, 0)
            s = proc_chunk(cs0, s, 0)

            def do_second(s):
                @pl.when(cs2 < rhi)
                def _pf2():
                    start_chunk(cs2, 0)

                wait_chunk(cs1, 1)
                return proc_chunk(cs1, s, 1)

            return lax.cond(cs1 < rhi, do_second, lambda t: t, s)

        return lax.cond(cs0 < rhi, do_pair, lambda t: t, s)

    NPAIRS = (N + CHUNK - 1) // CHUNK // 2 + 1
    lax.fori_loop(0, NPAIRS, pair_step, jnp.int32(0))

    pltpu.sync_copy(acc_v, out_hbm.at[pl.ds(sg * SEGT, SEGT), pl.ds(c0, CW)])


def _mlp_body(y2_ref, wt_ref, b_ref, out_ref):
    y = y2_ref[...]
    for i in range(LAYERS):
        yw = lax.dot_general(
            y, wt_ref[i], (((1,), (1,)), ((), ())),
            preferred_element_type=jnp.float32,
            precision=lax.Precision.HIGHEST,
        )
        y = jnp.maximum(yw + b_ref[i], 0.0)
    out_ref[...] = y


_tc_mlp = pl.pallas_call(
    _mlp_body,
    out_shape=jax.ShapeDtypeStruct((NUM_SEG, DIM), jnp.float32),
)

BLK = 400


def _bcast_body(x_ref, idx_ref, y_ref, o_ref):
    idxv = idx_ref[0, 0, :]
    onehot = (idxv[:, None]
              == lax.broadcasted_iota(jnp.int32, (BLK, NUM_SEG), 1)
              ).astype(jnp.float32)
    o_ref[...] = x_ref[...] + jnp.dot(onehot, y_ref[...],
                                      preferred_element_type=jnp.float32)


_tc_bcast = pl.pallas_call(
    _bcast_body,
    grid=(N // BLK,),
    in_specs=[
        pl.BlockSpec((BLK, DIM), lambda i: (i, 0)),
        pl.BlockSpec((1, 1, BLK), lambda i: (i, 0, 0)),
        pl.BlockSpec((NUM_SEG, DIM), lambda i: (0, 0)),
    ],
    out_specs=pl.BlockSpec((BLK, DIM), lambda i: (i, 0)),
    out_shape=jax.ShapeDtypeStruct((N, DIM), jnp.float32),
)


@functools.partial(
    pl.kernel,
    out_type=jax.ShapeDtypeStruct((N, DIM), jnp.float32),
    mesh=_mesh,
    scratch_types=[
        pltpu.VMEM((CH3, DIM), jnp.float32),
        pltpu.VMEM((CH3, DIM), jnp.float32),
        pltpu.VMEM((CH3, DIM), jnp.float32),
        pltpu.VMEM((CH3, DIM), jnp.float32),
        pltpu.VMEM((CH3,), jnp.int32),
        pltpu.VMEM((CH3,), jnp.int32),
        pltpu.SemaphoreType.DMA,
        pltpu.SemaphoreType.DMA,
        pltpu.SemaphoreType.DMA,
        pltpu.SemaphoreType.DMA,
        pltpu.SemaphoreType.DMA,
        pltpu.SemaphoreType.DMA,
    ],
)
def _sc_gather(x_hbm, idx_hbm, y_hbm, out_hbm,
               xr0, xr1, yr0, yr1, idxb0, idxb1,
               s_x0, s_x1, s_y0, s_y1, s_i0, s_i1):
    cid = lax.axis_index("c")
    sid = lax.axis_index("s")
    wid = sid * NC + cid

    xr = (xr0, xr1)
    yr = (yr0, yr1)
    idxb = (idxb0, idxb1)
    s_x = (s_x0, s_x1)
    s_y = (s_y0, s_y1)
    s_i = (s_i0, s_i1)

    def start_in(c, b):
        base = c * CH3
        pltpu.async_copy(idx_hbm.at[pl.ds(base, CH3)], idxb[b], s_i[b])
        pltpu.async_copy(x_hbm.at[pl.ds(base, CH3)], xr[b], s_x[b])

    def wait_in(c, b):
        base = c * CH3
        pltpu.make_async_copy(idx_hbm.at[pl.ds(base, CH3)], idxb[b],
                              s_i[b]).wait()
        pltpu.make_async_copy(x_hbm.at[pl.ds(base, CH3)], xr[b],
                              s_x[b]).wait()

    def process(c, b):
        pltpu.async_copy(y_hbm.at[idxb[b]], yr[b], s_y[b]).wait()

        def _add_row(r, carry):
            for j in range(DIM // LANES):
                sl = pl.ds(j * LANES, LANES)
                xr[b][r, sl] = xr[b][r, sl] + yr[b][r, sl]
            return carry

        lax.fori_loop(0, CH3, _add_row, 0)
        pltpu.sync_copy(xr[b], out_hbm.at[pl.ds(c * CH3, CH3)])

    start_in(wid, 0)
    for k in range(NFULL3):
        b = k % 2
        if k + 1 < NFULL3:
            start_in(wid + NW * (k + 1), (k + 1) % 2)
        else:
            @pl.when(wid < NTAIL3)
            def _pf_tail():
                start_in(wid + NW * NFULL3, NFULL3 % 2)
        wait_in(wid + NW * k, b)
        process(wid + NW * k, b)

    @pl.when(wid < NTAIL3)
    def _tail():
        wait_in(wid + NW * NFULL3, NFULL3 % 2)
        process(wid + NW * NFULL3, NFULL3 % 2)


def kernel(x, edge_features, edge_idx, batch_idx, W, b):
    del edge_features, edge_idx
    b3 = b[:, None, :]
    bnd = jnp.searchsorted(batch_idx, jnp.arange(NUM_SEG + 1, dtype=jnp.int32))
    bnd_pad = jnp.full((BND_PAD,), N, dtype=jnp.int32).at[:NUM_SEG + 1].set(
        bnd.astype(jnp.int32))
    zeros = jnp.zeros((SEGT, DIM), jnp.float32)
    ysum = _sc_scatter(x, bnd_pad, zeros, batch_idx)
    y = _tc_mlp(ysum, W, b3)
    return _tc_bcast(x, batch_idx.reshape(N // BLK, 1, BLK), y)


# speedup vs baseline: 1.1007x; 1.1007x over previous
import functools

import jax
import jax.numpy as jnp
from jax import lax
from jax.experimental import pallas as pl
from jax.experimental.pallas import tpu as pltpu
from jax.experimental.pallas import tpu_sc as plsc

N = 10000
DIM = 512
NUM_SEG = 512
LAYERS = 6
LANES = 16

NC = 2
NS = 16
NW = NC * NS

CHUNK = 320
SG = 8
CG = 4
CW = DIM // CG
SEGT = NUM_SEG // SG
BND_PAD = SG * SEGT + SEGT + LANES

CH3 = 40
NCH3 = N // CH3
NFULL3 = NCH3 // NW
NTAIL3 = NCH3 - NW * NFULL3

_mesh = plsc.VectorSubcoreMesh(core_axis_name="c", subcore_axis_name="s")


@functools.partial(
    pl.kernel,
    out_type=jax.ShapeDtypeStruct((NUM_SEG, DIM), jnp.float32),
    mesh=_mesh,
    scratch_types=[
        pltpu.VMEM((CHUNK, CW), jnp.float32),
        pltpu.VMEM((CHUNK, CW), jnp.float32),
        pltpu.VMEM((SEGT + LANES,), jnp.int32),
        pltpu.VMEM((CHUNK + LANES,), jnp.int32),
        pltpu.VMEM((CHUNK + LANES,), jnp.int32),
        pltpu.VMEM((SEGT, CW), jnp.float32),
        pltpu.SemaphoreType.DMA,
        pltpu.SemaphoreType.DMA,
        pltpu.SemaphoreType.DMA,
        pltpu.SemaphoreType.DMA,
        pltpu.SemaphoreType.DMA,
    ],
)
def _sc_scatter(x_hbm, bnd_hbm, zeros_hbm, idxf_hbm, out_hbm,
                rows_v0, rows_v1, bnd_v, idx_v0, idx_v1, acc_v,
                s_r0, s_r1, s_i0, s_i1, s_z):
    cid = lax.axis_index("c")
    sid = lax.axis_index("s")
    wid = sid * NC + cid
    sg = wid // CG
    cg = wid % CG
    c0 = cg * CW

    zc = pltpu.async_copy(zeros_hbm.at[pl.ds(0, SEGT), pl.ds(c0, CW)],
                          acc_v, s_z)
    pltpu.sync_copy(bnd_hbm.at[pl.ds(sg * SEGT, SEGT + LANES)], bnd_v)
    zc.wait()

    def bget(i):
        return bnd_v[pl.ds(i, LANES)][0]

    rlo = bget(0)
    rhi = bget(SEGT)
    start = (rlo // 8) * 8

    zacc = tuple(jnp.zeros((LANES,), jnp.float32) for _ in range(CW // LANES))

    rows_b = (rows_v0, rows_v1)
    idx_b = (idx_v0, idx_v1)
    s_r = (s_r0, s_r1)
    s_i = (s_i0, s_i1)

    def base_of(cstart):
        return (jnp.minimum(cstart, N - CHUNK) // 8) * 8

    def start_chunk(cstart, bi):
        bk = base_of(cstart)
        pltpu.async_copy(x_hbm.at[pl.ds(bk, CHUNK), pl.ds(c0, CW)],
                         rows_b[bi], s_r[bi])
        pltpu.async_copy(idxf_hbm.at[pl.ds(bk, CHUNK)],
                         idx_b[bi].at[pl.ds(0, CHUNK)], s_i[bi])

    def wait_chunk(cstart, bi):
        bk = base_of(cstart)
        pltpu.make_async_copy(x_hbm.at[pl.ds(bk, CHUNK), pl.ds(c0, CW)],
                              rows_b[bi], s_r[bi]).wait()
        pltpu.make_async_copy(idxf_hbm.at[pl.ds(bk, CHUNK)],
                              idx_b[bi].at[pl.ds(0, CHUNK)], s_i[bi]).wait()

    def add_range(accs, lo, hi, bk, bi):
        def body(r, a):
            return tuple(
                a[j] + rows_b[bi][r - bk, pl.ds(j * LANES, LANES)]
                for j in range(CW // LANES)
            )
        return lax.fori_loop(lo, hi, body, accs)

    def flush_add(s_local, accs):
        for j in range(CW // LANES):
            sl = pl.ds(j * LANES, LANES)
            acc_v[s_local, sl] = acc_v[s_local, sl] + accs[j]

    def proc_chunk(cstart, s, bi):
        bk = base_of(cstart)
        win_hi = jnp.minimum(cstart + CHUNK, rhi)

        s_hi = idx_b[bi][pl.ds(win_hi - 1 - bk, LANES)][0] - sg * SEGT

        def seg_body(s2, carry):
            accs = add_range(zacc, jnp.maximum(bget(s2), cstart),
                             bget(s2 + 1), bk, bi)
            flush_add(s2, accs)
            return carry

        lax.fori_loop(s, s_hi, seg_body, 0)
        accs3 = add_range(zacc, jnp.maximum(bget(s_hi), cstart), win_hi,
                          bk, bi)
        flush_add(jnp.clip(s_hi, 0, SEGT - 1), accs3)
        return s_hi

    @pl.when(start < rhi)
    def _prologue():
        start_chunk(start, 0)

    def pair_step(p, s):
        cs0 = start + (2 * p) * CHUNK
        cs1 = cs0 + CHUNK
        cs2 = cs1 + CHUNK

        def do_pair(s):
            @pl.when(cs1 < rhi)
            def _pf1():
                start_chunk(cs1, 1)

            wait_chunk(cs0, 0)
            s = proc_chunk(cs0, s, 0)

            def do_second(s):
                @pl.when(cs2 < rhi)
                def _pf2():
                    start_chunk(cs2, 0)

                wait_chunk(cs1, 1)
                return proc_chunk(cs1, s, 1)

            return lax.cond(cs1 < rhi, do_second, lambda t: t, s)

        return lax.cond(cs0 < rhi, do_pair, lambda t: t, s)

    NPAIRS = (N + CHUNK - 1) // CHUNK // 2 + 1
    lax.fori_loop(0, NPAIRS, pair_step, jnp.int32(0))

    pltpu.sync_copy(acc_v, out_hbm.at[pl.ds(sg * SEGT, SEGT), pl.ds(c0, CW)])


def _mlp_body(y2_ref, wt_ref, b_ref, out_ref):
    y = y2_ref[...]
    for i in range(LAYERS):
        yw = lax.dot_general(
            y, wt_ref[i], (((1,), (0,)), ((), ())),
            preferred_element_type=jnp.float32,
            precision=lax.Precision.HIGHEST,
        )
        y = jnp.maximum(yw + b_ref[i], 0.0)
    out_ref[...] = y


_tc_mlp = pl.pallas_call(
    _mlp_body,
    out_shape=jax.ShapeDtypeStruct((NUM_SEG, DIM), jnp.float32),
)

BLK = 400


def _bcast_body(x_ref, idx_ref, y_ref, o_ref):
    idxv = idx_ref[0, 0, :]
    onehot = (idxv[:, None]
              == lax.broadcasted_iota(jnp.int32, (BLK, NUM_SEG), 1)
              ).astype(jnp.float32)
    o_ref[...] = x_ref[...] + jnp.dot(onehot, y_ref[...],
                                      preferred_element_type=jnp.float32)


_tc_bcast = pl.pallas_call(
    _bcast_body,
    grid=(N // BLK,),
    in_specs=[
        pl.BlockSpec((BLK, DIM), lambda i: (i, 0)),
        pl.BlockSpec((1, 1, BLK), lambda i: (i, 0, 0)),
        pl.BlockSpec((NUM_SEG, DIM), lambda i: (0, 0)),
    ],
    out_specs=pl.BlockSpec((BLK, DIM), lambda i: (i, 0)),
    out_shape=jax.ShapeDtypeStruct((N, DIM), jnp.float32),
)


@functools.partial(
    pl.kernel,
    out_type=jax.ShapeDtypeStruct((N, DIM), jnp.float32),
    mesh=_mesh,
    scratch_types=[
        pltpu.VMEM((CH3, DIM), jnp.float32),
        pltpu.VMEM((CH3, DIM), jnp.float32),
        pltpu.VMEM((CH3, DIM), jnp.float32),
        pltpu.VMEM((CH3, DIM), jnp.float32),
        pltpu.VMEM((CH3,), jnp.int32),
        pltpu.VMEM((CH3,), jnp.int32),
        pltpu.SemaphoreType.DMA,
        pltpu.SemaphoreType.DMA,
        pltpu.SemaphoreType.DMA,
        pltpu.SemaphoreType.DMA,
        pltpu.SemaphoreType.DMA,
        pltpu.SemaphoreType.DMA,
    ],
)
def _sc_gather(x_hbm, idx_hbm, y_hbm, out_hbm,
               xr0, xr1, yr0, yr1, idxb0, idxb1,
               s_x0, s_x1, s_y0, s_y1, s_i0, s_i1):
    cid = lax.axis_index("c")
    sid = lax.axis_index("s")
    wid = sid * NC + cid

    xr = (xr0, xr1)
    yr = (yr0, yr1)
    idxb = (idxb0, idxb1)
    s_x = (s_x0, s_x1)
    s_y = (s_y0, s_y1)
    s_i = (s_i0, s_i1)

    def start_in(c, b):
        base = c * CH3
        pltpu.async_copy(idx_hbm.at[pl.ds(base, CH3)], idxb[b], s_i[b])
        pltpu.async_copy(x_hbm.at[pl.ds(base, CH3)], xr[b], s_x[b])

    def wait_in(c, b):
        base = c * CH3
        pltpu.make_async_copy(idx_hbm.at[pl.ds(base, CH3)], idxb[b],
                              s_i[b]).wait()
        pltpu.make_async_copy(x_hbm.at[pl.ds(base, CH3)], xr[b],
                              s_x[b]).wait()

    def process(c, b):
        pltpu.async_copy(y_hbm.at[idxb[b]], yr[b], s_y[b]).wait()

        def _add_row(r, carry):
            for j in range(DIM // LANES):
                sl = pl.ds(j * LANES, LANES)
                xr[b][r, sl] = xr[b][r, sl] + yr[b][r, sl]
            return carry

        lax.fori_loop(0, CH3, _add_row, 0)
        pltpu.sync_copy(xr[b], out_hbm.at[pl.ds(c * CH3, CH3)])

    start_in(wid, 0)
    for k in range(NFULL3):
        b = k % 2
        if k + 1 < NFULL3:
            start_in(wid + NW * (k + 1), (k + 1) % 2)
        else:
            @pl.when(wid < NTAIL3)
            def _pf_tail():
                start_in(wid + NW * NFULL3, NFULL3 % 2)
        wait_in(wid + NW * k, b)
        process(wid + NW * k, b)

    @pl.when(wid < NTAIL3)
    def _tail():
        wait_in(wid + NW * NFULL3, NFULL3 % 2)
        process(wid + NW * NFULL3, NFULL3 % 2)


def kernel(x, edge_features, edge_idx, batch_idx, W, b):
    del edge_features, edge_idx
    Wt = jnp.swapaxes(W, 1, 2)
    b3 = b[:, None, :]
    bnd = jnp.searchsorted(batch_idx, jnp.arange(NUM_SEG + 1, dtype=jnp.int32))
    bnd_pad = jnp.full((BND_PAD,), N, dtype=jnp.int32).at[:NUM_SEG + 1].set(
        bnd.astype(jnp.int32))
    zeros = jnp.zeros((SEGT, DIM), jnp.float32)
    ysum = _sc_scatter(x, bnd_pad, zeros, batch_idx)
    y = _tc_mlp(ysum, Wt, b3)
    return _tc_bcast(x, batch_idx.reshape(N // BLK, 1, BLK), y)


# speedup vs baseline: 1.1033x; 1.0024x over previous
import functools

import jax
import jax.numpy as jnp
from jax import lax
from jax.experimental import pallas as pl
from jax.experimental.pallas import tpu as pltpu
from jax.experimental.pallas import tpu_sc as plsc

N = 10000
DIM = 512
NUM_SEG = 512
LAYERS = 6
LANES = 16

NC = 2
NS = 16
NW = NC * NS

CHUNK = 160
SG = 8
CG = 4
CW = DIM // CG
SEGT = NUM_SEG // SG
BND_PAD = SG * SEGT + SEGT + LANES

_mesh = plsc.VectorSubcoreMesh(core_axis_name="c", subcore_axis_name="s")


@functools.partial(
    pl.kernel,
    out_type=jax.ShapeDtypeStruct((NUM_SEG, DIM), jnp.float32),
    mesh=_mesh,
    scratch_types=[
        pltpu.VMEM((CHUNK, CW), jnp.float32),
        pltpu.VMEM((CHUNK, CW), jnp.float32),
        pltpu.VMEM((SEGT + LANES,), jnp.int32),
        pltpu.VMEM((CHUNK + LANES,), jnp.int32),
        pltpu.VMEM((CHUNK + LANES,), jnp.int32),
        pltpu.VMEM((SEGT, CW), jnp.float32),
        pltpu.SemaphoreType.DMA,
        pltpu.SemaphoreType.DMA,
        pltpu.SemaphoreType.DMA,
        pltpu.SemaphoreType.DMA,
        pltpu.SemaphoreType.DMA,
    ],
)
def _sc_scatter(x_hbm, bnd_hbm, zeros_hbm, idxf_hbm, out_hbm,
                rows_v0, rows_v1, bnd_v, idx_v0, idx_v1, acc_v,
                s_r0, s_r1, s_i0, s_i1, s_z):
    cid = lax.axis_index("c")
    sid = lax.axis_index("s")
    wid = sid * NC + cid
    sg = wid // CG
    cg = wid % CG
    c0 = cg * CW

    zc = pltpu.async_copy(zeros_hbm.at[pl.ds(0, SEGT), pl.ds(c0, CW)],
                          acc_v, s_z)
    pltpu.sync_copy(bnd_hbm.at[pl.ds(sg * SEGT, SEGT + LANES)], bnd_v)
    zc.wait()

    def bget(i):
        return bnd_v[pl.ds(i, LANES)][0]

    rlo = bget(0)
    rhi = bget(SEGT)
    start = (rlo // 8) * 8

    zacc = tuple(jnp.zeros((LANES,), jnp.float32) for _ in range(CW // LANES))

    rows_b = (rows_v0, rows_v1)
    idx_b = (idx_v0, idx_v1)
    s_r = (s_r0, s_r1)
    s_i = (s_i0, s_i1)

    def base_of(cstart):
        return (jnp.minimum(cstart, N - CHUNK) // 8) * 8

    def start_chunk(cstart, bi):
        bk = base_of(cstart)
        pltpu.async_copy(x_hbm.at[pl.ds(bk, CHUNK), pl.ds(c0, CW)],
                         rows_b[bi], s_r[bi])
        pltpu.async_copy(idxf_hbm.at[pl.ds(bk, CHUNK)],
                         idx_b[bi].at[pl.ds(0, CHUNK)], s_i[bi])

    def wait_chunk(cstart, bi):
        bk = base_of(cstart)
        pltpu.make_async_copy(x_hbm.at[pl.ds(bk, CHUNK), pl.ds(c0, CW)],
                              rows_b[bi], s_r[bi]).wait()
        pltpu.make_async_copy(idxf_hbm.at[pl.ds(bk, CHUNK)],
                              idx_b[bi].at[pl.ds(0, CHUNK)], s_i[bi]).wait()

    def add_range(accs, lo, hi, bk, bi):
        def body(r, a):
            return tuple(
                a[j] + rows_b[bi][r - bk, pl.ds(j * LANES, LANES)]
                for j in range(CW // LANES)
            )
        return lax.fori_loop(lo, hi, body, accs)

    def flush_add(s_local, accs):
        for j in range(CW // LANES):
            sl = pl.ds(j * LANES, LANES)
            acc_v[s_local, sl] = acc_v[s_local, sl] + accs[j]

    def proc_chunk(cstart, s, bi):
        bk = base_of(cstart)
        win_hi = jnp.minimum(cstart + CHUNK, rhi)

        s_hi = idx_b[bi][pl.ds(win_hi - 1 - bk, LANES)][0] - sg * SEGT

        def seg_body(s2, carry):
            accs = add_range(zacc, jnp.maximum(bget(s2), cstart),
                             bget(s2 + 1), bk, bi)
            flush_add(s2, accs)
            return carry

        lax.fori_loop(s, s_hi, seg_body, 0)
        accs3 = add_range(zacc, jnp.maximum(bget(s_hi), cstart), win_hi,
                          bk, bi)
        flush_add(jnp.clip(s_hi, 0, SEGT - 1), accs3)
        return s_hi

    @pl.when(start < rhi)
    def _prologue():
        start_chunk(start, 0)

    def pair_step(p, s):
        cs0 = start + (2 * p) * CHUNK
        cs1 = cs0 + CHUNK
        cs2 = cs1 + CHUNK

        def do_pair(s):
            @pl.when(cs1 < rhi)
            def _pf1():
                start_chunk(cs1, 1)

            wait_chunk(cs0, 0)
            s = proc_chunk(cs0, s, 0)

            def do_second(s):
                @pl.when(cs2 < rhi)
                def _pf2():
                    start_chunk(cs2, 0)

                wait_chunk(cs1, 1)
                return proc_chunk(cs1, s, 1)

            return lax.cond(cs1 < rhi, do_second, lambda t: t, s)

        return lax.cond(cs0 < rhi, do_pair, lambda t: t, s)

    NPAIRS = (N + CHUNK - 1) // CHUNK // 2 + 1
    lax.fori_loop(0, NPAIRS, pair_step, jnp.int32(0))

    pltpu.sync_copy(acc_v, out_hbm.at[pl.ds(sg * SEGT, SEGT), pl.ds(c0, CW)])


def _mlp_body(y2_ref, wt_ref, b_ref, out_ref):
    y = y2_ref[...]
    for i in range(LAYERS):
        yw = lax.dot_general(
            y, wt_ref[i], (((1,), (0,)), ((), ())),
            preferred_element_type=jnp.float32,
            precision=lax.Precision.HIGHEST,
        )
        y = jnp.maximum(yw + b_ref[i], 0.0)
    out_ref[...] = y


_tc_mlp = pl.pallas_call(
    _mlp_body,
    out_shape=jax.ShapeDtypeStruct((NUM_SEG, DIM), jnp.float32),
)

BLK = 400


def _bcast_body(x_ref, idx_ref, y_ref, o_ref):
    idxv = idx_ref[0, 0, :]
    onehot = (idxv[:, None]
              == lax.broadcasted_iota(jnp.int32, (BLK, NUM_SEG), 1)
              ).astype(jnp.float32)
    o_ref[...] = x_ref[...] + jnp.dot(onehot, y_ref[...],
                                      preferred_element_type=jnp.float32)


_tc_bcast = pl.pallas_call(
    _bcast_body,
    grid=(N // BLK,),
    in_specs=[
        pl.BlockSpec((BLK, DIM), lambda i: (i, 0)),
        pl.BlockSpec((1, 1, BLK), lambda i: (i, 0, 0)),
        pl.BlockSpec((NUM_SEG, DIM), lambda i: (0, 0)),
    ],
    out_specs=pl.BlockSpec((BLK, DIM), lambda i: (i, 0)),
    out_shape=jax.ShapeDtypeStruct((N, DIM), jnp.float32),
)


def kernel(x, edge_features, edge_idx, batch_idx, W, b):
    del edge_features, edge_idx
    Wt = jnp.swapaxes(W, 1, 2)
    b3 = b[:, None, :]
    bnd = jnp.searchsorted(batch_idx, jnp.arange(NUM_SEG + 1, dtype=jnp.int32))
    bnd_pad = jnp.full((BND_PAD,), N, dtype=jnp.int32).at[:NUM_SEG + 1].set(
        bnd.astype(jnp.int32))
    zeros = jnp.zeros((SEGT, DIM), jnp.float32)
    ysum = _sc_scatter(x, bnd_pad, zeros, batch_idx)
    y = _tc_mlp(ysum, Wt, b3)
    return _tc_bcast(x, batch_idx.reshape(N // BLK, 1, BLK), y)
